# baseline (speedup 1.0000x reference)
import functools

import jax
import jax.numpy as jnp
from jax import lax
from jax.experimental import pallas as pl
from jax.experimental.pallas import tpu as pltpu
from jax.experimental.pallas import tpu_sc as plsc

BATCH = 1024
SEQ = 200
EMB = 64
NLANE = 16
NW = 32
PER_W = BATCH * SEQ // NW
C = 100
NCHUNK = PER_W // C
BLK = SEQ
NB = PER_W // BLK
GPB = BLK // C


def _sc_embed(idx2d, off2d, table2, pos):
    mesh = plsc.VectorSubcoreMesh(core_axis_name="c", subcore_axis_name="s")

    @functools.partial(
        pl.kernel,
        mesh=mesh,
        compiler_params=pltpu.CompilerParams(use_tc_tiling_on_sc=False),
        out_type=jax.ShapeDtypeStruct((BATCH * SEQ, EMB), jnp.float32),
        scratch_types=[
            pltpu.VMEM((NCHUNK, C), jnp.int32),
            pltpu.VMEM((NCHUNK, C), jnp.int32),
            pltpu.VMEM((BLK, 2 * EMB), jnp.float32),
            pltpu.VMEM((BLK, 2 * EMB), jnp.float32),
            pltpu.VMEM((BLK, EMB), jnp.float32),
            pltpu.VMEM((BLK, EMB), jnp.float32),
            pltpu.VMEM((SEQ, EMB), jnp.float32),
            pltpu.SemaphoreType.DMA,
            pltpu.SemaphoreType.DMA,
        ],
    )
    def k(idx_hbm, off_hbm, table_hbm, pos_hbm, out_hbm, idx_v, off_v,
          buf0, buf1, ob0, ob1, pos_v, gsem, osem):
        wid = lax.axis_index("s") * 2 + lax.axis_index("c")
        base = wid * PER_W
        pltpu.sync_copy(pos_hbm, pos_v)
        pltpu.sync_copy(idx_hbm.at[pl.ds(wid * NCHUNK, NCHUNK)], idx_v)
        pltpu.sync_copy(off_hbm.at[pl.ds(wid * NCHUNK, NCHUNK)], off_v)

        bufs = (buf0, buf1)
        obufs = (ob0, ob1)

        def fire_gathers(b, buf):
            return [
                pltpu.async_copy(
                    table_hbm.at[idx_v.at[b * GPB + j]],
                    buf.at[pl.ds(j * C, C)],
                    gsem,
                )
                for j in range(GPB)
            ]

        def add_pos(b, buf, obuf):
            def body(i, carry):
                off = off_v[b * GPB + i // C, pl.ds(i % C, 1)][0]
                for cc in range(EMB // NLANE):
                    sl = pl.ds(cc * NLANE, NLANE)
                    obuf[i, sl] = (
                        buf[i, pl.ds(off + cc * NLANE, NLANE)] + pos_v[i, sl])
                return carry

            lax.fori_loop(0, BLK, body, 0)

        gd = {0: fire_gathers(0, bufs[0])}
        od = {}
        for b in range(NB):
            q = b % 2
            for d in gd.pop(b):
                d.wait()
            if b + 1 < NB:
                gd[b + 1] = fire_gathers(b + 1, bufs[(b + 1) % 2])
            if b >= 2:
                od.pop(b - 2).wait()
            add_pos(b, bufs[q], obufs[q])
            od[b] = pltpu.async_copy(
                obufs[q], out_hbm.at[pl.ds(base + b * BLK, BLK)], osem)
        od.pop(NB - 2).wait()
        od.pop(NB - 1).wait()

    return k(idx2d, off2d, table2, pos)


def kernel(x, tok_emb, pos_emb):
    flat = x.reshape(NW * NCHUNK, C)
    idx2d = lax.shift_right_logical(flat, 1)
    off2d = lax.shift_left(jnp.bitwise_and(flat, 1), 6)
    table2 = tok_emb.reshape(tok_emb.shape[0] // 2, 2 * EMB)
    pos = pos_emb[0, :SEQ, :]
    out = _sc_embed(idx2d, off2d, table2, pos)
    return out.reshape(BATCH, SEQ, EMB)


# speedup vs baseline: 1.0956x; 1.0956x over previous
import functools

import jax
import jax.numpy as jnp
from jax import lax
from jax.experimental import pallas as pl
from jax.experimental.pallas import tpu as pltpu
from jax.experimental.pallas import tpu_sc as plsc

BATCH = 1024
SEQ = 200
EMB = 64
NLANE = 16
NW = 32
PER_W = BATCH * SEQ // NW
C = 100
NCHUNK = PER_W // C
BLK = SEQ
NB = PER_W // BLK
GPB = BLK // C


def _sc_embed(idx2d, off2d, table2, pos):
    mesh = plsc.VectorSubcoreMesh(core_axis_name="c", subcore_axis_name="s")

    @functools.partial(
        pl.kernel,
        mesh=mesh,
        compiler_params=pltpu.CompilerParams(use_tc_tiling_on_sc=False),
        out_type=jax.ShapeDtypeStruct((BATCH * SEQ, EMB), jnp.float32),
        scratch_types=[
            pltpu.VMEM((NCHUNK, C), jnp.int32),
            pltpu.VMEM((NB + 1, BLK), jnp.int32),
            pltpu.VMEM((BLK, 2 * EMB), jnp.float32),
            pltpu.VMEM((BLK, 2 * EMB), jnp.float32),
            pltpu.VMEM((BLK, EMB), jnp.float32),
            pltpu.VMEM((BLK, EMB), jnp.float32),
            pltpu.VMEM((SEQ, EMB), jnp.float32),
            pltpu.SemaphoreType.DMA,
            pltpu.SemaphoreType.DMA,
            pltpu.SemaphoreType.DMA,
            pltpu.SemaphoreType.DMA,
        ],
    )
    def k(idx_hbm, off_hbm, table_hbm, pos_hbm, out_hbm, idx_v, off_v,
          buf0, buf1, ob0, ob1, pos_v, gsem0, gsem1, osem0, osem1):
        wid = lax.axis_index("s") * 2 + lax.axis_index("c")
        base = wid * PER_W
        pltpu.sync_copy(pos_hbm, pos_v)
        pltpu.sync_copy(idx_hbm.at[pl.ds(wid * NCHUNK, NCHUNK)], idx_v)
        pltpu.sync_copy(off_hbm.at[pl.ds(wid * NB, NB)],
                        off_v.at[pl.ds(0, NB)])

        def fire_gathers(b, buf, sem):
            for j in range(GPB):
                pltpu.async_copy(
                    table_hbm.at[idx_v.at[b * GPB + j]],
                    buf.at[pl.ds(j * C, C)],
                    sem,
                )

        def wait_gathers(b, buf, sem):
            for j in range(GPB):
                pltpu.make_async_copy(
                    table_hbm.at[idx_v.at[b * GPB + j]],
                    buf.at[pl.ds(j * C, C)],
                    sem,
                ).wait()

        def fire_out(b, obuf, sem):
            pltpu.async_copy(
                obuf, out_hbm.at[pl.ds(base + b * BLK, BLK)], sem)

        def wait_out(b, obuf, sem):
            pltpu.make_async_copy(
                obuf, out_hbm.at[pl.ds(base + b * BLK, BLK)], sem).wait()

        def add_pos(b, buf, obuf):
            def body(g, carry):
                s = g * 8
                offs = off_v[b, pl.ds(s, NLANE)]
                for j in range(8):
                    off = offs[j]
                    r = s + j
                    for cc in range(EMB // NLANE):
                        sl = pl.ds(cc * NLANE, NLANE)
                        obuf[r, sl] = (
                            buf[r, pl.ds(off + cc * NLANE, NLANE)]
                            + pos_v[r, sl])
                return carry

            lax.fori_loop(0, BLK // 8, body, 0)

        fire_gathers(0, buf0, gsem0)
        fire_gathers(1, buf1, gsem1)

        wait_gathers(0, buf0, gsem0)
        add_pos(0, buf0, ob0)
        fire_out(0, ob0, osem0)
        fire_gathers(2, buf0, gsem0)
        wait_gathers(1, buf1, gsem1)
        add_pos(1, buf1, ob1)
        fire_out(1, ob1, osem1)
        fire_gathers(3, buf1, gsem1)

        def pair(p, carry):
            b0 = 2 * p
            wait_gathers(b0, buf0, gsem0)
            wait_out(b0 - 2, ob0, osem0)
            add_pos(b0, buf0, ob0)
            fire_out(b0, ob0, osem0)
            fire_gathers(b0 + 2, buf0, gsem0)
            wait_gathers(b0 + 1, buf1, gsem1)
            wait_out(b0 - 1, ob1, osem1)
            add_pos(b0 + 1, buf1, ob1)
            fire_out(b0 + 1, ob1, osem1)
            fire_gathers(b0 + 3, buf1, gsem1)
            return carry

        lax.fori_loop(1, NB // 2 - 1, pair, 0)

        b0 = NB - 2
        wait_gathers(b0, buf0, gsem0)
        wait_out(b0 - 2, ob0, osem0)
        add_pos(b0, buf0, ob0)
        fire_out(b0, ob0, osem0)
        wait_gathers(b0 + 1, buf1, gsem1)
        wait_out(b0 - 1, ob1, osem1)
        add_pos(b0 + 1, buf1, ob1)
        fire_out(b0 + 1, ob1, osem1)
        wait_out(b0, ob0, osem0)
        wait_out(b0 + 1, ob1, osem1)

    return k(idx2d, off2d, table2, pos)


def kernel(x, tok_emb, pos_emb):
    flat = x.reshape(NW * NCHUNK, C)
    idx2d = lax.shift_right_logical(flat, 1)
    off2d = lax.shift_left(jnp.bitwise_and(flat, 1), 6).reshape(
        NW * NB, BLK)
    table2 = tok_emb.reshape(tok_emb.shape[0] // 2, 2 * EMB)
    pos = pos_emb[0, :SEQ, :]
    out = _sc_embed(idx2d, off2d, table2, pos)
    return out.reshape(BATCH, SEQ, EMB)
